# baseline (speedup 1.0000x reference)
import jax
import jax.numpy as jnp
from jax.experimental import pallas as pl

ALPHA = 0.1


def _body(adj_ref, x_ref, h_ref, o_ref):
    acc = jnp.dot(adj_ref[...], x_ref[...], preferred_element_type=jnp.float32)
    o_ref[...] = (1.0 - ALPHA) * acc + ALPHA * h_ref[...]


def kernel(x, adj, h):
    n, d = x.shape
    bm = 512
    out = pl.pallas_call(
        _body,
        grid=(n // bm,),
        in_specs=[
            pl.BlockSpec((bm, n), lambda i: (i, 0)),
            pl.BlockSpec((n, d), lambda i: (0, 0)),
            pl.BlockSpec((bm, d), lambda i: (i, 0)),
        ],
        out_specs=pl.BlockSpec((bm, d), lambda i: (i, 0)),
        out_shape=jax.ShapeDtypeStruct((n, d), jnp.float32),
    )(adj, x, h)
    return out
